# Initial kernel scaffold; baseline (speedup 1.0000x reference)
import jax
import jax.numpy as jnp
from jax.experimental import pallas as pl


def kernel(seq_len, pos_embedding):
    raise NotImplementedError("write your pallas kernel here")


# speedup vs baseline: 1.5909x; 1.5909x over previous
import functools

import jax
import jax.numpy as jnp
from jax import lax
from jax.experimental import pallas as pl
from jax.experimental.pallas import tpu as pltpu
from jax.experimental.pallas import tpu_sc as plsc

MAX_SEQ_LEN = 8192
EMBED_DIM = 1024

_NC = 2
_NS = 16
_NW = _NC * _NS
_ROWS_PER_W = MAX_SEQ_LEN // _NW
_CHUNK = 32
_NCHUNKS = _ROWS_PER_W // _CHUNK


def _make_sc_gather():
    mesh = plsc.VectorSubcoreMesh(core_axis_name="c", subcore_axis_name="s")

    @functools.partial(
        pl.kernel,
        mesh=mesh,
        out_type=jax.ShapeDtypeStruct((MAX_SEQ_LEN, EMBED_DIM), jnp.float32),
        scratch_types=[
            pltpu.VMEM((_ROWS_PER_W,), jnp.int32),
            pltpu.VMEM((_CHUNK, EMBED_DIM), jnp.float32),
            pltpu.VMEM((_CHUNK, EMBED_DIM), jnp.float32),
            pltpu.SemaphoreType.DMA,
            pltpu.SemaphoreType.DMA,
        ],
    )
    def gather_kernel(idx_hbm, table_hbm, out_hbm, idx_v, buf0, buf1, sem0, sem1):
        wid = lax.axis_index("s") * _NC + lax.axis_index("c")
        base = wid * _ROWS_PER_W
        pltpu.sync_copy(idx_hbm.at[pl.ds(base, _ROWS_PER_W)], idx_v)
        bufs = (buf0, buf1)
        sems = (sem0, sem1)
        copies = [None] * _NCHUNKS
        copies[0] = pltpu.async_copy(
            table_hbm.at[idx_v.at[pl.ds(0, _CHUNK)]], bufs[0], sems[0])
        for g in range(_NCHUNKS):
            if g + 1 < _NCHUNKS:
                copies[g + 1] = pltpu.async_copy(
                    table_hbm.at[idx_v.at[pl.ds((g + 1) * _CHUNK, _CHUNK)]],
                    bufs[(g + 1) % 2], sems[(g + 1) % 2])
            copies[g].wait()
            pltpu.sync_copy(bufs[g % 2],
                            out_hbm.at[pl.ds(base + g * _CHUNK, _CHUNK)])

    return gather_kernel


_sc_gather = _make_sc_gather()


def kernel(seq_len, pos_embedding):
    positions = jnp.arange(MAX_SEQ_LEN, dtype=jnp.int32) % jnp.asarray(
        seq_len, jnp.int32)
    return _sc_gather(positions, pos_embedding)
